# baseline (speedup 1.0000x reference)
import jax
import jax.numpy as jnp
from jax import lax
from jax.experimental import pallas as pl
from jax.experimental.pallas import tpu as pltpu
from jax.experimental.pallas import tpu_sc as plsc

N = 10000
E = 320000
IN_FEATS = 128
OUT_FEATS = 16
EDGE_FEATS = 4
HEADS = 8
ROW = 144
NC, NS = 2, 16
NW = NC * NS
C = 80
EPT = E // NW
NCHUNK = EPT // C
NROWCHUNK = N // C


def _node_tables_body(h_ref, wm_ref, ws_ref, wd_ref, om_ref, os_ref, od_ref):
    x = h_ref[...]
    om_ref[...] = jnp.dot(x, wm_ref[...], preferred_element_type=jnp.float32)
    os_ref[...] = jnp.dot(x, ws_ref[...], preferred_element_type=jnp.float32)
    od_ref[...] = jnp.dot(x, wd_ref[...], preferred_element_type=jnp.float32)


def _node_tables(h, wm, ws, wd):
    bn = 1000
    return pl.pallas_call(
        _node_tables_body,
        grid=(N // bn,),
        in_specs=[
            pl.BlockSpec((bn, IN_FEATS), lambda i: (i, 0)),
            pl.BlockSpec((IN_FEATS, 128), lambda i: (0, 0)),
            pl.BlockSpec((IN_FEATS, 16), lambda i: (0, 0)),
            pl.BlockSpec((IN_FEATS, 16), lambda i: (0, 0)),
        ],
        out_specs=[
            pl.BlockSpec((bn, 128), lambda i: (i, 0)),
            pl.BlockSpec((bn, 16), lambda i: (i, 0)),
            pl.BlockSpec((bn, 16), lambda i: (i, 0)),
        ],
        out_shape=[
            jax.ShapeDtypeStruct((N, 128), jnp.float32),
            jax.ShapeDtypeStruct((N, 16), jnp.float32),
            jax.ShapeDtypeStruct((N, 16), jnp.float32),
        ],
    )(h, wm, ws, wd)


def _edge_table_body(ef_ref, wm_ref, wt_ref, om_ref, ot_ref):
    x = ef_ref[...]
    om_ref[...] = jnp.dot(x, wm_ref[...], preferred_element_type=jnp.float32)
    ot_ref[...] = jnp.dot(x, wt_ref[...], preferred_element_type=jnp.float32)


def _edge_tables(edge_feat, wm, wt):
    be = 8000
    return pl.pallas_call(
        _edge_table_body,
        grid=(E // be,),
        in_specs=[
            pl.BlockSpec((be, EDGE_FEATS), lambda i: (i, 0)),
            pl.BlockSpec((EDGE_FEATS, 128), lambda i: (0, 0)),
            pl.BlockSpec((EDGE_FEATS, 16), lambda i: (0, 0)),
        ],
        out_specs=[
            pl.BlockSpec((be, 128), lambda i: (i, 0)),
            pl.BlockSpec((be, 16), lambda i: (i, 0)),
        ],
        out_shape=[
            jax.ShapeDtypeStruct((E, 128), jnp.float32),
            jax.ShapeDtypeStruct((E, 16), jnp.float32),
        ],
    )(edge_feat, wm, wt)


def _edge_pass_body(mnode_hbm, tsrc_hbm, tdst_hbm, src_hbm, dst_hbm,
                    etm_hbm, et16_hbm, num_hbm, den_hbm,
                    idx_v, srow_v, gs_v, gd_v, etm_v, et_v, num_v, acc_sh):
    cid = lax.axis_index("c")
    sid = lax.axis_index("s")
    wid = sid * NC + cid

    @pl.loop(0, C)
    def _zrow(r):
        @pl.loop(0, ROW, step=16)
        def _zcol(k):
            num_v[r, pl.ds(k, 16)] = jnp.zeros((16,), jnp.float32)

    @pl.loop(sid, NROWCHUNK, step=NS)
    def _zacc(j):
        pltpu.sync_copy(num_v, acc_sh.at[pl.ds(j * C, C)])

    plsc.subcore_barrier()

    base0 = wid * EPT

    @pl.loop(0, NCHUNK)
    def _chunk(i):
        base = base0 + i * C
        pltpu.sync_copy(src_hbm.at[pl.ds(base, C)], idx_v.at[0])
        pltpu.sync_copy(dst_hbm.at[pl.ds(base, C)], idx_v.at[1])
        pltpu.sync_copy(mnode_hbm.at[idx_v.at[0]], srow_v)
        pltpu.sync_copy(tsrc_hbm.at[idx_v.at[0]], gs_v)
        pltpu.sync_copy(tdst_hbm.at[idx_v.at[1]], gd_v)
        pltpu.sync_copy(etm_hbm.at[pl.ds(base, C)], etm_v)
        pltpu.sync_copy(et16_hbm.at[pl.ds(base, C)], et_v)

        @pl.loop(0, C)
        def _edge(c):
            a = gs_v[c, :] + gd_v[c, :] + et_v[c, :]
            a = jnp.where(a >= 0.0, a, a * 0.2)
            p = jnp.exp(a)
            num_v[c, pl.ds(128, 16)] = p
            for k in range(HEADS):
                num_v[c, pl.ds(16 * k, 16)] = (
                    srow_v[c, pl.ds(16 * k, 16)]
                    + etm_v[c, pl.ds(16 * k, 16)]) * p[k]

        pltpu.sync_copy(num_v, acc_sh.at[idx_v.at[1]], add=True)

    plsc.subcore_barrier()

    @pl.loop(sid, NROWCHUNK, step=NS)
    def _wb(j):
        pltpu.sync_copy(acc_sh.at[pl.ds(j * C, C), pl.ds(0, 128)],
                        num_hbm.at[cid, pl.ds(j * C, C)])
        pltpu.sync_copy(acc_sh.at[pl.ds(j * C, C), pl.ds(128, 16)],
                        den_hbm.at[cid, pl.ds(j * C, C)])


def _edge_pass(mnode, tsrc, tdst, src, dst, etm, et16):
    mesh = plsc.VectorSubcoreMesh(core_axis_name="c", subcore_axis_name="s")
    f = pl.kernel(
        _edge_pass_body,
        out_type=(
            jax.ShapeDtypeStruct((NC, N, 128), jnp.float32),
            jax.ShapeDtypeStruct((NC, N, 16), jnp.float32),
        ),
        mesh=mesh,
        compiler_params=pltpu.CompilerParams(use_tc_tiling_on_sc=False),
        scratch_types=[
            pltpu.VMEM((2, C), jnp.int32),
            pltpu.VMEM((C, 128), jnp.float32),
            pltpu.VMEM((C, 16), jnp.float32),
            pltpu.VMEM((C, 16), jnp.float32),
            pltpu.VMEM((C, 128), jnp.float32),
            pltpu.VMEM((C, 16), jnp.float32),
            pltpu.VMEM((C, ROW), jnp.float32),
            pltpu.VMEM_SHARED((N, ROW), jnp.float32),
        ],
    )
    return f(mnode, tsrc, tdst, src, dst, etm, et16)


def _finalize_body(num_ref, den_ref, o_ref):
    r = num_ref[0] + num_ref[1]
    d = den_ref[0] + den_ref[1]
    d = jnp.maximum(d, 1e-12)
    acc = jnp.zeros(o_ref.shape, jnp.float32)
    for h in range(HEADS):
        acc = acc + r[:, 16 * h:16 * h + 16] / d[:, h:h + 1]
    o_ref[...] = acc * (1.0 / HEADS)


def _finalize(num, den):
    bn = 1000
    return pl.pallas_call(
        _finalize_body,
        grid=(N // bn,),
        in_specs=[
            pl.BlockSpec((NC, bn, 128), lambda i: (0, i, 0)),
            pl.BlockSpec((NC, bn, 16), lambda i: (0, i, 0)),
        ],
        out_specs=pl.BlockSpec((bn, 16), lambda i: (i, 0)),
        out_shape=jax.ShapeDtypeStruct((N, 16), jnp.float32),
    )(num, den)


def kernel(h, edge_index, edge_feat, W_node, W_edge, attention_src,
           attention_dst, W_msg):
    f32 = jnp.float32
    wn3 = W_node.reshape(IN_FEATS, HEADS, OUT_FEATS)
    a_src = jnp.einsum("jhk,hk->jh", wn3, attention_src)
    a_dst = jnp.einsum("jhk,hk->jh", wn3, attention_dst)
    pad_n = jnp.zeros((IN_FEATS, 8), f32)
    pad_e = jnp.zeros((EDGE_FEATS, 8), f32)
    ws = jnp.concatenate([a_src, pad_n], axis=1)
    wd = jnp.concatenate([a_dst, pad_n], axis=1)
    wt = jnp.concatenate([W_edge, pad_e], axis=1)

    mnode, tsrc, tdst = _node_tables(h, W_msg[:IN_FEATS], ws, wd)
    etm, et16 = _edge_tables(edge_feat, W_msg[IN_FEATS:], wt)
    num, den = _edge_pass(mnode, tsrc, tdst, edge_index[0], edge_index[1],
                          etm, et16)
    return _finalize(num, den)


# speedup vs baseline: 1.5821x; 1.5821x over previous
import jax
import jax.numpy as jnp
from jax import lax
from jax.experimental import pallas as pl
from jax.experimental.pallas import tpu as pltpu
from jax.experimental.pallas import tpu_sc as plsc

N = 10000
E = 320000
IN_FEATS = 128
OUT_FEATS = 16
EDGE_FEATS = 4
HEADS = 8
ROW = 144
NC, NS = 2, 16
NW = NC * NS
C = 40
EPT = E // NW
NCHUNK = EPT // C
NROWCHUNK = N // C


def _node_tables_body(h_ref, w1_ref, wd_ref, o1_ref, od_ref):
    x = h_ref[...]
    o1_ref[...] = jnp.dot(x, w1_ref[...], preferred_element_type=jnp.float32)
    od_ref[...] = jnp.dot(x, wd_ref[...], preferred_element_type=jnp.float32)


def _node_tables(h, w1, wd):
    bn = 1000
    return pl.pallas_call(
        _node_tables_body,
        grid=(N // bn,),
        in_specs=[
            pl.BlockSpec((bn, IN_FEATS), lambda i: (i, 0)),
            pl.BlockSpec((IN_FEATS, ROW), lambda i: (0, 0)),
            pl.BlockSpec((IN_FEATS, 16), lambda i: (0, 0)),
        ],
        out_specs=[
            pl.BlockSpec((bn, ROW), lambda i: (i, 0)),
            pl.BlockSpec((bn, 16), lambda i: (i, 0)),
        ],
        out_shape=[
            jax.ShapeDtypeStruct((N, ROW), jnp.float32),
            jax.ShapeDtypeStruct((N, 16), jnp.float32),
        ],
    )(h, w1, wd)


def _edge_table_body(ef_ref, wm_ref, wt_ref, om_ref, ot_ref):
    x = ef_ref[...]
    om_ref[...] = jnp.dot(x, wm_ref[...], preferred_element_type=jnp.float32)
    ot_ref[...] = jnp.dot(x, wt_ref[...], preferred_element_type=jnp.float32)


def _edge_tables(edge_feat, wm, wt):
    be = 8000
    return pl.pallas_call(
        _edge_table_body,
        grid=(E // be,),
        in_specs=[
            pl.BlockSpec((be, EDGE_FEATS), lambda i: (i, 0)),
            pl.BlockSpec((EDGE_FEATS, 128), lambda i: (0, 0)),
            pl.BlockSpec((EDGE_FEATS, 16), lambda i: (0, 0)),
        ],
        out_specs=[
            pl.BlockSpec((be, 128), lambda i: (i, 0)),
            pl.BlockSpec((be, 16), lambda i: (i, 0)),
        ],
        out_shape=[
            jax.ShapeDtypeStruct((E, 128), jnp.float32),
            jax.ShapeDtypeStruct((E, 16), jnp.float32),
        ],
    )(edge_feat, wm, wt)


def _edge_pass_body(tsrc_hbm, tdst_hbm, src3_hbm, dst3_hbm,
                    etm_hbm, et16_hbm, num_hbm, den_hbm,
                    idx4, srow2, gd2, etm2, et2, num2, acc_sh,
                    isem, gsem, lsem, ssem):
    cid = lax.axis_index("c")
    sid = lax.axis_index("s")
    wid = sid * NC + cid
    base0 = wid * EPT

    @pl.loop(0, C)
    def _zrow(r):
        @pl.loop(0, ROW, step=16)
        def _zcol(k):
            num2[0, r, pl.ds(k, 16)] = jnp.zeros((16,), jnp.float32)

    @pl.loop(sid, NROWCHUNK, step=NS)
    def _zacc(j):
        pltpu.sync_copy(num2.at[0], acc_sh.at[pl.ds(j * C, C)])

    plsc.subcore_barrier()

    def _issue_idx(i, sync=False):
        r = lax.rem(i, 4)
        if sync:
            pltpu.sync_copy(src3_hbm.at[wid, i], idx4.at[r, 0])
            pltpu.sync_copy(dst3_hbm.at[wid, i], idx4.at[r, 1])
        else:
            pltpu.async_copy(src3_hbm.at[wid, i], idx4.at[r, 0], isem)
            pltpu.async_copy(dst3_hbm.at[wid, i], idx4.at[r, 1], isem)

    def _drain_idx(i):
        r = lax.rem(i, 4)
        pltpu.make_async_copy(src3_hbm.at[wid, i], idx4.at[r, 0],
                              isem).wait()
        pltpu.make_async_copy(dst3_hbm.at[wid, i], idx4.at[r, 1],
                              isem).wait()

    def _issue_data(i, b):
        base = base0 + i * C
        r = lax.rem(i, 4)
        pltpu.async_copy(etm_hbm.at[pl.ds(base, C)], etm2.at[b], lsem.at[b])
        pltpu.async_copy(et16_hbm.at[pl.ds(base, C)], et2.at[b], lsem.at[b])
        pltpu.async_copy(tsrc_hbm.at[idx4.at[r, 0]], srow2.at[b],
                         gsem.at[b])
        pltpu.async_copy(tdst_hbm.at[idx4.at[r, 1]], gd2.at[b], gsem.at[b])

    def _drain_data(i, b):
        base = base0 + i * C
        r = lax.rem(i, 4)
        pltpu.make_async_copy(etm_hbm.at[pl.ds(base, C)], etm2.at[b],
                              lsem.at[b]).wait()
        pltpu.make_async_copy(et16_hbm.at[pl.ds(base, C)], et2.at[b],
                              lsem.at[b]).wait()
        pltpu.make_async_copy(tsrc_hbm.at[idx4.at[r, 0]], srow2.at[b],
                              gsem.at[b]).wait()
        pltpu.make_async_copy(tdst_hbm.at[idx4.at[r, 1]], gd2.at[b],
                              gsem.at[b]).wait()

    def _drain_scatter(j, b):
        pltpu.make_async_copy(num2.at[b], acc_sh.at[idx4.at[lax.rem(j, 4), 1]],
                              ssem.at[b]).wait()

    _issue_idx(0, sync=True)
    _issue_data(0, 0)
    _issue_idx(1)

    @pl.loop(0, NCHUNK)
    def _chunk(i):
        b = lax.rem(i, 2)

        @pl.when(i + 1 < NCHUNK)
        def _():
            _drain_idx(i + 1)
            _issue_data(i + 1, 1 - b)

        @pl.when(i >= 2)
        def _():
            _drain_scatter(i - 2, b)

        @pl.when(i + 2 < NCHUNK)
        def _():
            _issue_idx(i + 2)

        _drain_data(i, b)

        @pl.loop(0, C)
        def _edge(c):
            a = srow2[b, c, pl.ds(128, 16)] + gd2[b, c, :] + et2[b, c, :]
            a = jnp.where(a >= 0.0, a, a * 0.2)
            p = jnp.exp(a)
            num2[b, c, pl.ds(128, 16)] = p
            for k in range(HEADS):
                num2[b, c, pl.ds(16 * k, 16)] = (
                    srow2[b, c, pl.ds(16 * k, 16)]
                    + etm2[b, c, pl.ds(16 * k, 16)]) * p[k]

        pltpu.async_copy(num2.at[b], acc_sh.at[idx4.at[lax.rem(i, 4), 1]],
                         ssem.at[b], add=True)

    _drain_scatter(NCHUNK - 2, lax.rem(NCHUNK - 2, 2))
    _drain_scatter(NCHUNK - 1, lax.rem(NCHUNK - 1, 2))

    plsc.subcore_barrier()

    @pl.loop(sid, NROWCHUNK, step=NS)
    def _wb(j):
        pltpu.sync_copy(acc_sh.at[pl.ds(j * C, C), pl.ds(0, 128)],
                        num_hbm.at[cid, pl.ds(j * C, C)])
        pltpu.sync_copy(acc_sh.at[pl.ds(j * C, C), pl.ds(128, 16)],
                        den_hbm.at[cid, pl.ds(j * C, C)])


def _edge_pass(tsrc, tdst, src3, dst3, etm, et16):
    mesh = plsc.VectorSubcoreMesh(core_axis_name="c", subcore_axis_name="s")
    f = pl.kernel(
        _edge_pass_body,
        out_type=(
            jax.ShapeDtypeStruct((NC, N, 128), jnp.float32),
            jax.ShapeDtypeStruct((NC, N, 16), jnp.float32),
        ),
        mesh=mesh,
        compiler_params=pltpu.CompilerParams(use_tc_tiling_on_sc=False),
        scratch_types=[
            pltpu.VMEM((4, 2, C), jnp.int32),
            pltpu.VMEM((2, C, ROW), jnp.float32),
            pltpu.VMEM((2, C, 16), jnp.float32),
            pltpu.VMEM((2, C, 128), jnp.float32),
            pltpu.VMEM((2, C, 16), jnp.float32),
            pltpu.VMEM((2, C, ROW), jnp.float32),
            pltpu.VMEM_SHARED((N, ROW), jnp.float32),
            pltpu.SemaphoreType.DMA,
            pltpu.SemaphoreType.DMA((2,)),
            pltpu.SemaphoreType.DMA((2,)),
            pltpu.SemaphoreType.DMA((2,)),
        ],
    )
    return f(tsrc, tdst, src3, dst3, etm, et16)


def _finalize_body(num_ref, den_ref, o_ref):
    r = num_ref[0] + num_ref[1]
    d = den_ref[0] + den_ref[1]
    d = jnp.maximum(d, 1e-12)
    acc = jnp.zeros(o_ref.shape, jnp.float32)
    for h in range(HEADS):
        acc = acc + r[:, 16 * h:16 * h + 16] / d[:, h:h + 1]
    o_ref[...] = acc * (1.0 / HEADS)


def _finalize(num, den):
    bn = 1000
    return pl.pallas_call(
        _finalize_body,
        grid=(N // bn,),
        in_specs=[
            pl.BlockSpec((NC, bn, 128), lambda i: (0, i, 0)),
            pl.BlockSpec((NC, bn, 16), lambda i: (0, i, 0)),
        ],
        out_specs=pl.BlockSpec((bn, 16), lambda i: (i, 0)),
        out_shape=jax.ShapeDtypeStruct((N, 16), jnp.float32),
    )(num, den)


def kernel(h, edge_index, edge_feat, W_node, W_edge, attention_src,
           attention_dst, W_msg):
    f32 = jnp.float32
    wn3 = W_node.reshape(IN_FEATS, HEADS, OUT_FEATS)
    a_src = jnp.einsum("jhk,hk->jh", wn3, attention_src)
    a_dst = jnp.einsum("jhk,hk->jh", wn3, attention_dst)
    pad_n = jnp.zeros((IN_FEATS, 8), f32)
    pad_e = jnp.zeros((EDGE_FEATS, 8), f32)
    w1 = jnp.concatenate([W_msg[:IN_FEATS], a_src, pad_n], axis=1)
    wd = jnp.concatenate([a_dst, pad_n], axis=1)
    wt = jnp.concatenate([W_edge, pad_e], axis=1)

    tsrc, tdst = _node_tables(h, w1, wd)
    etm, et16 = _edge_tables(edge_feat, W_msg[IN_FEATS:], wt)
    src3 = edge_index[0].reshape(NW, NCHUNK, C)
    dst3 = edge_index[1].reshape(NW, NCHUNK, C)
    num, den = _edge_pass(tsrc, tdst, src3, dst3, etm, et16)
    return _finalize(num, den)


# speedup vs baseline: 1.6943x; 1.0709x over previous
import jax
import jax.numpy as jnp
from jax import lax
from jax.experimental import pallas as pl
from jax.experimental.pallas import tpu as pltpu
from jax.experimental.pallas import tpu_sc as plsc

N = 10000
E = 320000
IN_FEATS = 128
OUT_FEATS = 16
EDGE_FEATS = 4
HEADS = 8
ROW = 144
NC, NS = 2, 16
NW = NC * NS
C = 40
EPT = E // NW
NCHUNK = EPT // C
NROWCHUNK = N // C


def _node_tables_body(h_ref, w1_ref, wd_ref, o1_ref, od_ref):
    x = h_ref[...]
    o1_ref[...] = jnp.dot(x, w1_ref[...], preferred_element_type=jnp.float32)
    od_ref[...] = jnp.dot(x, wd_ref[...], preferred_element_type=jnp.float32)


def _node_tables(h, w1, wd):
    bn = 1000
    return pl.pallas_call(
        _node_tables_body,
        grid=(N // bn,),
        in_specs=[
            pl.BlockSpec((bn, IN_FEATS), lambda i: (i, 0)),
            pl.BlockSpec((IN_FEATS, ROW), lambda i: (0, 0)),
            pl.BlockSpec((IN_FEATS, 16), lambda i: (0, 0)),
        ],
        out_specs=[
            pl.BlockSpec((bn, ROW), lambda i: (i, 0)),
            pl.BlockSpec((bn, 16), lambda i: (i, 0)),
        ],
        out_shape=[
            jax.ShapeDtypeStruct((N, ROW), jnp.float32),
            jax.ShapeDtypeStruct((N, 16), jnp.float32),
        ],
    )(h, w1, wd)


def _edge_table_body(eft_ref, efr8_ref, wm_ref, wb_ref, om_ref, ot_ref):
    dn = (((0,), (0,)), ((), ()))
    om_ref[...] = lax.dot_general(eft_ref[...], wm_ref[...], dn,
                                  preferred_element_type=jnp.float32)
    ot_ref[...] = jnp.dot(efr8_ref[...], wb_ref[...],
                          preferred_element_type=jnp.float32)


def _edge_tables(edge_feat_t, edge_feat_r8, wm, wbig):
    be = 6400
    return pl.pallas_call(
        _edge_table_body,
        grid=(E // be,),
        in_specs=[
            pl.BlockSpec((EDGE_FEATS, be), lambda i: (0, i)),
            pl.BlockSpec((be // 8, 32), lambda i: (i, 0)),
            pl.BlockSpec((EDGE_FEATS, 128), lambda i: (0, 0)),
            pl.BlockSpec((32, 128), lambda i: (0, 0)),
        ],
        out_specs=[
            pl.BlockSpec((be, 128), lambda i: (i, 0)),
            pl.BlockSpec((be // 8, 128), lambda i: (i, 0)),
        ],
        out_shape=[
            jax.ShapeDtypeStruct((E, 128), jnp.float32),
            jax.ShapeDtypeStruct((E // 8, 128), jnp.float32),
        ],
    )(edge_feat_t, edge_feat_r8, wm, wbig)


def _edge_pass_body(tsrc_hbm, tdst_hbm, src3_hbm, dst3_hbm,
                    etm_hbm, etp_hbm, num_hbm, den_hbm,
                    idx4, srow2, gd2, etm2, et2, num2, acc_sh,
                    isem, gsem, lsem, ssem):
    cid = lax.axis_index("c")
    sid = lax.axis_index("s")
    wid = sid * NC + cid
    base0 = wid * EPT

    @pl.loop(0, C)
    def _zrow(r):
        @pl.loop(0, ROW, step=16)
        def _zcol(k):
            num2[0, r, pl.ds(k, 16)] = jnp.zeros((16,), jnp.float32)

    @pl.loop(sid, NROWCHUNK, step=NS)
    def _zacc(j):
        pltpu.sync_copy(num2.at[0], acc_sh.at[pl.ds(j * C, C)])

    plsc.subcore_barrier()

    def _issue_idx(i, sync=False):
        r = lax.rem(i, 4)
        if sync:
            pltpu.sync_copy(src3_hbm.at[wid, i], idx4.at[r, 0])
            pltpu.sync_copy(dst3_hbm.at[wid, i], idx4.at[r, 1])
        else:
            pltpu.async_copy(src3_hbm.at[wid, i], idx4.at[r, 0], isem)
            pltpu.async_copy(dst3_hbm.at[wid, i], idx4.at[r, 1], isem)

    def _drain_idx(i):
        r = lax.rem(i, 4)
        pltpu.make_async_copy(src3_hbm.at[wid, i], idx4.at[r, 0],
                              isem).wait()
        pltpu.make_async_copy(dst3_hbm.at[wid, i], idx4.at[r, 1],
                              isem).wait()

    def _issue_data(i, b):
        base = base0 + i * C
        r = lax.rem(i, 4)
        pltpu.async_copy(etm_hbm.at[pl.ds(base, C)], etm2.at[b], lsem.at[b])
        pltpu.async_copy(etp_hbm.at[pl.ds(base // 8, C // 8)], et2.at[b],
                         lsem.at[b])
        pltpu.async_copy(tsrc_hbm.at[idx4.at[r, 0]], srow2.at[b],
                         gsem.at[b])
        pltpu.async_copy(tdst_hbm.at[idx4.at[r, 1]], gd2.at[b], gsem.at[b])

    def _drain_data(i, b):
        base = base0 + i * C
        r = lax.rem(i, 4)
        pltpu.make_async_copy(etm_hbm.at[pl.ds(base, C)], etm2.at[b],
                              lsem.at[b]).wait()
        pltpu.make_async_copy(etp_hbm.at[pl.ds(base // 8, C // 8)],
                              et2.at[b], lsem.at[b]).wait()
        pltpu.make_async_copy(tsrc_hbm.at[idx4.at[r, 0]], srow2.at[b],
                              gsem.at[b]).wait()
        pltpu.make_async_copy(tdst_hbm.at[idx4.at[r, 1]], gd2.at[b],
                              gsem.at[b]).wait()

    def _drain_scatter(j, b):
        pltpu.make_async_copy(num2.at[b], acc_sh.at[idx4.at[lax.rem(j, 4), 1]],
                              ssem.at[b]).wait()

    _issue_idx(0, sync=True)
    _issue_data(0, 0)
    _issue_idx(1)

    @pl.loop(0, NCHUNK)
    def _chunk(i):
        b = lax.rem(i, 2)

        @pl.when(i + 1 < NCHUNK)
        def _():
            _drain_idx(i + 1)
            _issue_data(i + 1, 1 - b)

        @pl.when(i >= 2)
        def _():
            _drain_scatter(i - 2, b)

        @pl.when(i + 2 < NCHUNK)
        def _():
            _issue_idx(i + 2)

        _drain_data(i, b)

        @pl.loop(0, C, unroll=4)
        def _edge(c):
            et = et2[b, c // 8, pl.ds(lax.rem(c, 8) * 16, 16)]
            a = srow2[b, c, pl.ds(128, 16)] + gd2[b, c, :] + et
            a = jnp.where(a >= 0.0, a, a * 0.2)
            p = jnp.exp(a)
            num2[b, c, pl.ds(128, 16)] = p
            for k in range(HEADS):
                num2[b, c, pl.ds(16 * k, 16)] = (
                    srow2[b, c, pl.ds(16 * k, 16)]
                    + etm2[b, c, pl.ds(16 * k, 16)]) * p[k]

        pltpu.async_copy(num2.at[b], acc_sh.at[idx4.at[lax.rem(i, 4), 1]],
                         ssem.at[b], add=True)

    _drain_scatter(NCHUNK - 2, lax.rem(NCHUNK - 2, 2))
    _drain_scatter(NCHUNK - 1, lax.rem(NCHUNK - 1, 2))

    plsc.subcore_barrier()

    @pl.loop(sid, NROWCHUNK, step=NS)
    def _wb(j):
        pltpu.sync_copy(acc_sh.at[pl.ds(j * C, C), pl.ds(0, 128)],
                        num_hbm.at[cid, pl.ds(j * C, C)])
        pltpu.sync_copy(acc_sh.at[pl.ds(j * C, C), pl.ds(128, 16)],
                        den_hbm.at[cid, pl.ds(j * C, C)])


def _edge_pass(tsrc, tdst, src3, dst3, etm, etp):
    mesh = plsc.VectorSubcoreMesh(core_axis_name="c", subcore_axis_name="s")
    f = pl.kernel(
        _edge_pass_body,
        out_type=(
            jax.ShapeDtypeStruct((NC, N, 128), jnp.float32),
            jax.ShapeDtypeStruct((NC, N, 16), jnp.float32),
        ),
        mesh=mesh,
        compiler_params=pltpu.CompilerParams(use_tc_tiling_on_sc=False),
        scratch_types=[
            pltpu.VMEM((4, 2, C), jnp.int32),
            pltpu.VMEM((2, C, ROW), jnp.float32),
            pltpu.VMEM((2, C, 16), jnp.float32),
            pltpu.VMEM((2, C, 128), jnp.float32),
            pltpu.VMEM((2, C // 8, 128), jnp.float32),
            pltpu.VMEM((2, C, ROW), jnp.float32),
            pltpu.VMEM_SHARED((N, ROW), jnp.float32),
            pltpu.SemaphoreType.DMA,
            pltpu.SemaphoreType.DMA((2,)),
            pltpu.SemaphoreType.DMA((2,)),
            pltpu.SemaphoreType.DMA((2,)),
        ],
    )
    return f(tsrc, tdst, src3, dst3, etm, etp)


def _finalize_body(num_ref, den_ref, o_ref):
    r = num_ref[0] + num_ref[1]
    d = den_ref[0] + den_ref[1]
    d = jnp.maximum(d, 1e-12)
    acc = jnp.zeros(o_ref.shape, jnp.float32)
    for h in range(HEADS):
        acc = acc + r[:, 16 * h:16 * h + 16] / d[:, h:h + 1]
    o_ref[...] = acc * (1.0 / HEADS)


def _finalize(num, den):
    bn = 1000
    return pl.pallas_call(
        _finalize_body,
        grid=(N // bn,),
        in_specs=[
            pl.BlockSpec((NC, bn, 128), lambda i: (0, i, 0)),
            pl.BlockSpec((NC, bn, 16), lambda i: (0, i, 0)),
        ],
        out_specs=pl.BlockSpec((bn, 16), lambda i: (i, 0)),
        out_shape=jax.ShapeDtypeStruct((N, 16), jnp.float32),
    )(num, den)


def kernel(h, edge_index, edge_feat, W_node, W_edge, attention_src,
           attention_dst, W_msg):
    f32 = jnp.float32
    wn3 = W_node.reshape(IN_FEATS, HEADS, OUT_FEATS)
    a_src = jnp.einsum("jhk,hk->jh", wn3, attention_src)
    a_dst = jnp.einsum("jhk,hk->jh", wn3, attention_dst)
    pad_n = jnp.zeros((IN_FEATS, 8), f32)
    pad_e = jnp.zeros((EDGE_FEATS, 8), f32)
    w1 = jnp.concatenate([W_msg[:IN_FEATS], a_src, pad_n], axis=1)
    wd = jnp.concatenate([a_dst, pad_n], axis=1)
    wt = jnp.concatenate([W_edge, pad_e], axis=1)
    wbig = jnp.kron(jnp.eye(8, dtype=f32), wt)

    tsrc, tdst = _node_tables(h, w1, wd)
    etm, etp = _edge_tables(edge_feat.T, edge_feat.reshape(E // 8, 32),
                            W_msg[IN_FEATS:], wbig)
    src3 = edge_index[0].reshape(NW, NCHUNK, C)
    dst3 = edge_index[1].reshape(NW, NCHUNK, C)
    num, den = _edge_pass(tsrc, tdst, src3, dst3, etm, etp)
    return _finalize(num, den)


# speedup vs baseline: 3.7509x; 2.2138x over previous
import jax
import jax.numpy as jnp
from jax import lax
from jax.experimental import pallas as pl
from jax.experimental.pallas import tpu as pltpu
from jax.experimental.pallas import tpu_sc as plsc

N = 10000
E = 320000
IN_FEATS = 128
OUT_FEATS = 16
EDGE_FEATS = 4
HEADS = 8
NC, NS = 2, 16
NW = NC * NS
C = 80
EPT = E // NW
NCHUNK = EPT // C
NROWCHUNK = N // C

_GDN = lax.GatherDimensionNumbers(offset_dims=(), collapsed_slice_dims=(0,),
                                  start_index_map=(0,))
_INB = lax.GatherScatterMode.PROMISE_IN_BOUNDS


def _bcast_lane(vec, lane):
    idx = jnp.full((16, 1), lane, jnp.int32)
    return lax.gather(vec, idx, _GDN, (1,), mode=_INB)


def _node_tables_body(h_ref, wm_ref, ws_ref, wd_ref, om_ref, os_ref, od_ref):
    x = h_ref[...]
    om_ref[...] = jnp.dot(x, wm_ref[...], preferred_element_type=jnp.float32)
    os_ref[...] = jnp.dot(x, ws_ref[...], preferred_element_type=jnp.float32)
    od_ref[...] = jnp.dot(x, wd_ref[...], preferred_element_type=jnp.float32)


def _node_tables(h, wm, ws, wd):
    bn = 1000
    return pl.pallas_call(
        _node_tables_body,
        grid=(N // bn,),
        in_specs=[
            pl.BlockSpec((bn, IN_FEATS), lambda i: (i, 0)),
            pl.BlockSpec((IN_FEATS, 128), lambda i: (0, 0)),
            pl.BlockSpec((IN_FEATS, 16), lambda i: (0, 0)),
            pl.BlockSpec((IN_FEATS, 16), lambda i: (0, 0)),
        ],
        out_specs=[
            pl.BlockSpec((bn, 128), lambda i: (i, 0)),
            pl.BlockSpec((bn, 16), lambda i: (i, 0)),
            pl.BlockSpec((bn, 16), lambda i: (i, 0)),
        ],
        out_shape=[
            jax.ShapeDtypeStruct((N, 128), jnp.float32),
            jax.ShapeDtypeStruct((N, 16), jnp.float32),
            jax.ShapeDtypeStruct((N, 16), jnp.float32),
        ],
    )(h, wm, ws, wd)


def _edge_table_body(eft_ref, wm_ref, om_ref):
    dn = (((0,), (0,)), ((), ()))
    om_ref[...] = lax.dot_general(eft_ref[...], wm_ref[...], dn,
                                  preferred_element_type=jnp.float32)


def _edge_table(edge_feat_t, wm):
    be = 6400
    return pl.pallas_call(
        _edge_table_body,
        grid=(E // be,),
        in_specs=[
            pl.BlockSpec((EDGE_FEATS, be), lambda i: (0, i)),
            pl.BlockSpec((EDGE_FEATS, 128), lambda i: (0, 0)),
        ],
        out_specs=pl.BlockSpec((be, 128), lambda i: (i, 0)),
        out_shape=jax.ShapeDtypeStruct((E, 128), jnp.float32),
    )(edge_feat_t, wm)


def _den_pass_body(ts_hbm, td_hbm, eft_hbm, wt_hbm, src3_hbm, dst3_hbm,
                   p_hbm, den_hbm,
                   idx4, gs2, gd2, ef2, pv2, wtv, den_sh,
                   isem, gsem, lsem, ssem, wsem):
    cid = lax.axis_index("c")
    sid = lax.axis_index("s")
    wid = sid * NC + cid
    base0 = wid * EPT

    pltpu.sync_copy(wt_hbm, wtv)

    @pl.loop(0, C)
    def _zrow(r):
        pv2[0, r, :] = jnp.zeros((16,), jnp.float32)

    @pl.loop(sid, NROWCHUNK, step=NS)
    def _zacc(j):
        pltpu.sync_copy(pv2.at[0], den_sh.at[pl.ds(j * C, C)])

    plsc.subcore_barrier()

    def _issue_idx(i, sync=False):
        r = lax.rem(i, 4)
        if sync:
            pltpu.sync_copy(src3_hbm.at[wid, i], idx4.at[r, 0])
            pltpu.sync_copy(dst3_hbm.at[wid, i], idx4.at[r, 1])
        else:
            pltpu.async_copy(src3_hbm.at[wid, i], idx4.at[r, 0], isem)
            pltpu.async_copy(dst3_hbm.at[wid, i], idx4.at[r, 1], isem)

    def _drain_idx(i):
        r = lax.rem(i, 4)
        pltpu.make_async_copy(src3_hbm.at[wid, i], idx4.at[r, 0],
                              isem).wait()
        pltpu.make_async_copy(dst3_hbm.at[wid, i], idx4.at[r, 1],
                              isem).wait()

    def _issue_data(i, b):
        base = base0 + i * C
        r = lax.rem(i, 4)
        pltpu.async_copy(eft_hbm.at[:, pl.ds(base, C)], ef2.at[b],
                         lsem.at[b])
        pltpu.async_copy(ts_hbm.at[idx4.at[r, 0]], gs2.at[b], gsem.at[b])
        pltpu.async_copy(td_hbm.at[idx4.at[r, 1]], gd2.at[b], gsem.at[b])

    def _drain_data(i, b):
        base = base0 + i * C
        r = lax.rem(i, 4)
        pltpu.make_async_copy(eft_hbm.at[:, pl.ds(base, C)], ef2.at[b],
                              lsem.at[b]).wait()
        pltpu.make_async_copy(ts_hbm.at[idx4.at[r, 0]], gs2.at[b],
                              gsem.at[b]).wait()
        pltpu.make_async_copy(td_hbm.at[idx4.at[r, 1]], gd2.at[b],
                              gsem.at[b]).wait()

    def _drain_out(j, b):
        base = base0 + j * C
        pltpu.make_async_copy(pv2.at[b], den_sh.at[idx4.at[lax.rem(j, 4), 1]],
                              ssem.at[b]).wait()
        pltpu.make_async_copy(pv2.at[b], p_hbm.at[pl.ds(base, C)],
                              wsem.at[b]).wait()

    _issue_idx(0, sync=True)
    _issue_data(0, 0)
    _issue_idx(1)

    @pl.loop(0, NCHUNK)
    def _chunk(i):
        b = lax.rem(i, 2)

        @pl.when(i + 1 < NCHUNK)
        def _():
            _drain_idx(i + 1)
            _issue_data(i + 1, 1 - b)

        @pl.when(i >= 2)
        def _():
            _drain_out(i - 2, b)

        @pl.when(i + 2 < NCHUNK)
        def _():
            _issue_idx(i + 2)

        _drain_data(i, b)

        w0 = wtv[0, :]
        w1 = wtv[1, :]
        w2 = wtv[2, :]
        w3 = wtv[3, :]

        @pl.loop(0, C, step=16)
        def _grp(c0):
            v0 = ef2[b, 0, pl.ds(c0, 16)]
            v1 = ef2[b, 1, pl.ds(c0, 16)]
            v2 = ef2[b, 2, pl.ds(c0, 16)]
            v3 = ef2[b, 3, pl.ds(c0, 16)]
            for l in range(16):
                c = c0 + l
                et = (_bcast_lane(v0, l) * w0 + _bcast_lane(v1, l) * w1
                      + _bcast_lane(v2, l) * w2 + _bcast_lane(v3, l) * w3)
                a = gs2[b, c, :] + gd2[b, c, :] + et
                a = jnp.where(a >= 0.0, a, a * 0.2)
                pv2[b, c, :] = jnp.exp(a)

        base = base0 + i * C
        pltpu.async_copy(pv2.at[b], den_sh.at[idx4.at[lax.rem(i, 4), 1]],
                         ssem.at[b], add=True)
        pltpu.async_copy(pv2.at[b], p_hbm.at[pl.ds(base, C)], wsem.at[b])

    _drain_out(NCHUNK - 2, lax.rem(NCHUNK - 2, 2))
    _drain_out(NCHUNK - 1, lax.rem(NCHUNK - 1, 2))

    plsc.subcore_barrier()

    @pl.loop(sid, NROWCHUNK, step=NS)
    def _wb(j):
        pltpu.sync_copy(den_sh.at[pl.ds(j * C, C)],
                        den_hbm.at[cid, pl.ds(j * C, C)])


def _den_pass(ts16, td16, eft, wt, src3, dst3):
    mesh = plsc.VectorSubcoreMesh(core_axis_name="c", subcore_axis_name="s")
    f = pl.kernel(
        _den_pass_body,
        out_type=(
            jax.ShapeDtypeStruct((E, 16), jnp.float32),
            jax.ShapeDtypeStruct((NC, N, 16), jnp.float32),
        ),
        mesh=mesh,
        compiler_params=pltpu.CompilerParams(use_tc_tiling_on_sc=False),
        scratch_types=[
            pltpu.VMEM((4, 2, C), jnp.int32),
            pltpu.VMEM((2, C, 16), jnp.float32),
            pltpu.VMEM((2, C, 16), jnp.float32),
            pltpu.VMEM((2, EDGE_FEATS, C), jnp.float32),
            pltpu.VMEM((2, C, 16), jnp.float32),
            pltpu.VMEM((EDGE_FEATS, 16), jnp.float32),
            pltpu.VMEM_SHARED((N, 16), jnp.float32),
            pltpu.SemaphoreType.DMA,
            pltpu.SemaphoreType.DMA((2,)),
            pltpu.SemaphoreType.DMA((2,)),
            pltpu.SemaphoreType.DMA((2,)),
            pltpu.SemaphoreType.DMA((2,)),
        ],
    )
    return f(ts16, td16, eft, wt, src3, dst3)


def _combine_body(den_ref, o_ref):
    r = den_ref[0] + den_ref[1]
    o_ref[...] = 1.0 / (HEADS * jnp.maximum(r, 1e-12))


def _combine(den):
    bn = 1000
    return pl.pallas_call(
        _combine_body,
        grid=(N // bn,),
        in_specs=[pl.BlockSpec((NC, bn, 16), lambda i: (0, i, 0))],
        out_specs=pl.BlockSpec((bn, 16), lambda i: (i, 0)),
        out_shape=jax.ShapeDtypeStruct((N, 16), jnp.float32),
    )(den)


def _agg_pass_body(mn_hbm, di_hbm, etm_hbm, p_hbm, src3_hbm, dst3_hbm,
                   out_hbm,
                   idx4, srow2, gdi2, etm2, pp2, v2, acc_sh,
                   isem, gsem, lsem, ssem):
    cid = lax.axis_index("c")
    sid = lax.axis_index("s")
    wid = sid * NC + cid
    base0 = wid * EPT

    @pl.loop(0, C)
    def _zrow(r):
        v2[0, r, :] = jnp.zeros((16,), jnp.float32)

    @pl.loop(sid, NROWCHUNK, step=NS)
    def _zacc(j):
        pltpu.sync_copy(v2.at[0], acc_sh.at[pl.ds(j * C, C)])

    plsc.subcore_barrier()

    def _issue_idx(i, sync=False):
        r = lax.rem(i, 4)
        if sync:
            pltpu.sync_copy(src3_hbm.at[wid, i], idx4.at[r, 0])
            pltpu.sync_copy(dst3_hbm.at[wid, i], idx4.at[r, 1])
        else:
            pltpu.async_copy(src3_hbm.at[wid, i], idx4.at[r, 0], isem)
            pltpu.async_copy(dst3_hbm.at[wid, i], idx4.at[r, 1], isem)

    def _drain_idx(i):
        r = lax.rem(i, 4)
        pltpu.make_async_copy(src3_hbm.at[wid, i], idx4.at[r, 0],
                              isem).wait()
        pltpu.make_async_copy(dst3_hbm.at[wid, i], idx4.at[r, 1],
                              isem).wait()

    def _issue_data(i, b):
        base = base0 + i * C
        r = lax.rem(i, 4)
        pltpu.async_copy(etm_hbm.at[pl.ds(base, C)], etm2.at[b], lsem.at[b])
        pltpu.async_copy(p_hbm.at[pl.ds(base, C)], pp2.at[b], lsem.at[b])
        pltpu.async_copy(mn_hbm.at[idx4.at[r, 0]], srow2.at[b], gsem.at[b])
        pltpu.async_copy(di_hbm.at[idx4.at[r, 1]], gdi2.at[b], gsem.at[b])

    def _drain_data(i, b):
        base = base0 + i * C
        r = lax.rem(i, 4)
        pltpu.make_async_copy(etm_hbm.at[pl.ds(base, C)], etm2.at[b],
                              lsem.at[b]).wait()
        pltpu.make_async_copy(p_hbm.at[pl.ds(base, C)], pp2.at[b],
                              lsem.at[b]).wait()
        pltpu.make_async_copy(mn_hbm.at[idx4.at[r, 0]], srow2.at[b],
                              gsem.at[b]).wait()
        pltpu.make_async_copy(di_hbm.at[idx4.at[r, 1]], gdi2.at[b],
                              gsem.at[b]).wait()

    def _drain_scatter(j, b):
        pltpu.make_async_copy(v2.at[b], acc_sh.at[idx4.at[lax.rem(j, 4), 1]],
                              ssem.at[b]).wait()

    _issue_idx(0, sync=True)
    _issue_data(0, 0)
    _issue_idx(1)

    @pl.loop(0, NCHUNK)
    def _chunk(i):
        b = lax.rem(i, 2)

        @pl.when(i + 1 < NCHUNK)
        def _():
            _drain_idx(i + 1)
            _issue_data(i + 1, 1 - b)

        @pl.when(i >= 2)
        def _():
            _drain_scatter(i - 2, b)

        @pl.when(i + 2 < NCHUNK)
        def _():
            _issue_idx(i + 2)

        _drain_data(i, b)

        @pl.loop(0, C, unroll=2)
        def _edge(c):
            al = pp2[b, c, :] * gdi2[b, c, :]
            acc = (srow2[b, c, pl.ds(0, 16)]
                   + etm2[b, c, pl.ds(0, 16)]) * al[0]
            for k in range(1, HEADS):
                acc = acc + (srow2[b, c, pl.ds(16 * k, 16)]
                             + etm2[b, c, pl.ds(16 * k, 16)]) * al[k]
            v2[b, c, :] = acc

        pltpu.async_copy(v2.at[b], acc_sh.at[idx4.at[lax.rem(i, 4), 1]],
                         ssem.at[b], add=True)

    _drain_scatter(NCHUNK - 2, lax.rem(NCHUNK - 2, 2))
    _drain_scatter(NCHUNK - 1, lax.rem(NCHUNK - 1, 2))

    plsc.subcore_barrier()

    @pl.loop(sid, NROWCHUNK, step=NS)
    def _wb(j):
        pltpu.sync_copy(acc_sh.at[pl.ds(j * C, C)],
                        out_hbm.at[cid, pl.ds(j * C, C)])


def _agg_pass(mnode, dinv, etm, pbuf, src3, dst3):
    mesh = plsc.VectorSubcoreMesh(core_axis_name="c", subcore_axis_name="s")
    f = pl.kernel(
        _agg_pass_body,
        out_type=jax.ShapeDtypeStruct((NC, N, 16), jnp.float32),
        mesh=mesh,
        compiler_params=pltpu.CompilerParams(use_tc_tiling_on_sc=False),
        scratch_types=[
            pltpu.VMEM((4, 2, C), jnp.int32),
            pltpu.VMEM((2, C, 128), jnp.float32),
            pltpu.VMEM((2, C, 16), jnp.float32),
            pltpu.VMEM((2, C, 128), jnp.float32),
            pltpu.VMEM((2, C, 16), jnp.float32),
            pltpu.VMEM((2, C, 16), jnp.float32),
            pltpu.VMEM_SHARED((N, 16), jnp.float32),
            pltpu.SemaphoreType.DMA,
            pltpu.SemaphoreType.DMA((2,)),
            pltpu.SemaphoreType.DMA((2,)),
            pltpu.SemaphoreType.DMA((2,)),
        ],
    )
    return f(mnode, dinv, etm, pbuf, src3, dst3)


def _final_body(a_ref, o_ref):
    o_ref[...] = a_ref[0] + a_ref[1]


def _finalize(outp):
    bn = 1000
    return pl.pallas_call(
        _final_body,
        grid=(N // bn,),
        in_specs=[pl.BlockSpec((NC, bn, 16), lambda i: (0, i, 0))],
        out_specs=pl.BlockSpec((bn, 16), lambda i: (i, 0)),
        out_shape=jax.ShapeDtypeStruct((N, 16), jnp.float32),
    )(outp)


def kernel(h, edge_index, edge_feat, W_node, W_edge, attention_src,
           attention_dst, W_msg):
    f32 = jnp.float32
    wn3 = W_node.reshape(IN_FEATS, HEADS, OUT_FEATS)
    a_src = jnp.einsum("jhk,hk->jh", wn3, attention_src)
    a_dst = jnp.einsum("jhk,hk->jh", wn3, attention_dst)
    pad_n = jnp.zeros((IN_FEATS, 8), f32)
    pad_e = jnp.zeros((EDGE_FEATS, 8), f32)
    ws = jnp.concatenate([a_src, pad_n], axis=1)
    wd = jnp.concatenate([a_dst, pad_n], axis=1)
    wt = jnp.concatenate([W_edge, pad_e], axis=1)

    mnode, ts16, td16 = _node_tables(h, W_msg[:IN_FEATS], ws, wd)
    eft = edge_feat.T
    etm = _edge_table(eft, W_msg[IN_FEATS:])
    src3 = edge_index[0].reshape(NW, NCHUNK, C)
    dst3 = edge_index[1].reshape(NW, NCHUNK, C)
    pbuf, den = _den_pass(ts16, td16, eft, wt, src3, dst3)
    dinv = _combine(den)
    outp = _agg_pass(mnode, dinv, etm, pbuf, src3, dst3)
    return _finalize(outp)


# speedup vs baseline: 3.7514x; 1.0001x over previous
import jax
import jax.numpy as jnp
from jax import lax
from jax.experimental import pallas as pl
from jax.experimental.pallas import tpu as pltpu
from jax.experimental.pallas import tpu_sc as plsc

N = 10000
E = 320000
IN_FEATS = 128
OUT_FEATS = 16
EDGE_FEATS = 4
HEADS = 8
NC, NS = 2, 16
NW = NC * NS
C = 80
EPT = E // NW
NCHUNK = EPT // C
NROWCHUNK = N // C

_GDN = lax.GatherDimensionNumbers(offset_dims=(), collapsed_slice_dims=(0,),
                                  start_index_map=(0,))
_INB = lax.GatherScatterMode.PROMISE_IN_BOUNDS


def _bcast_lane(vec, lane):
    idx = jnp.full((16, 1), lane, jnp.int32)
    return lax.gather(vec, idx, _GDN, (1,), mode=_INB)


def _node_tables_body(h_ref, wm_ref, ws_ref, wd_ref, om_ref, os_ref, od_ref):
    x = h_ref[...]
    om_ref[...] = jnp.dot(x, wm_ref[...], preferred_element_type=jnp.float32)
    os_ref[...] = jnp.dot(x, ws_ref[...], preferred_element_type=jnp.float32)
    od_ref[...] = jnp.dot(x, wd_ref[...], preferred_element_type=jnp.float32)


def _node_tables(h, wm, ws, wd):
    bn = 1000
    return pl.pallas_call(
        _node_tables_body,
        grid=(N // bn,),
        in_specs=[
            pl.BlockSpec((bn, IN_FEATS), lambda i: (i, 0)),
            pl.BlockSpec((IN_FEATS, 128), lambda i: (0, 0)),
            pl.BlockSpec((IN_FEATS, 16), lambda i: (0, 0)),
            pl.BlockSpec((IN_FEATS, 16), lambda i: (0, 0)),
        ],
        out_specs=[
            pl.BlockSpec((bn, 128), lambda i: (i, 0)),
            pl.BlockSpec((bn, 16), lambda i: (i, 0)),
            pl.BlockSpec((bn, 16), lambda i: (i, 0)),
        ],
        out_shape=[
            jax.ShapeDtypeStruct((N, 128), jnp.float32),
            jax.ShapeDtypeStruct((N, 16), jnp.float32),
            jax.ShapeDtypeStruct((N, 16), jnp.float32),
        ],
    )(h, wm, ws, wd)


def _edge_table_body(eft_ref, wm_ref, om_ref):
    dn = (((0,), (0,)), ((), ()))
    om_ref[...] = lax.dot_general(eft_ref[...], wm_ref[...], dn,
                                  preferred_element_type=jnp.float32)


def _edge_table(edge_feat_t, wm):
    be = 6400
    return pl.pallas_call(
        _edge_table_body,
        grid=(E // be,),
        in_specs=[
            pl.BlockSpec((EDGE_FEATS, be), lambda i: (0, i)),
            pl.BlockSpec((EDGE_FEATS, 128), lambda i: (0, 0)),
        ],
        out_specs=pl.BlockSpec((be, 128), lambda i: (i, 0)),
        out_shape=jax.ShapeDtypeStruct((E, 128), jnp.float32),
    )(edge_feat_t, wm)


def _den_pass_body(ts_hbm, td_hbm, eft_hbm, wt_hbm, src3_hbm, dst3_hbm,
                   p_hbm, den_hbm,
                   idx4, gs2, gd2, ef2, pv2, wtv, den_sh,
                   isem, gsem, lsem, ssem, wsem):
    cid = lax.axis_index("c")
    sid = lax.axis_index("s")
    wid = sid * NC + cid
    base0 = wid * EPT

    pltpu.sync_copy(wt_hbm, wtv)

    @pl.loop(0, C)
    def _zrow(r):
        pv2[0, r, :] = jnp.zeros((16,), jnp.float32)

    @pl.loop(sid, NROWCHUNK, step=NS)
    def _zacc(j):
        pltpu.sync_copy(pv2.at[0], den_sh.at[pl.ds(j * C, C)])

    plsc.subcore_barrier()

    def _issue_idx(i, sync=False):
        r = lax.rem(i, 4)
        if sync:
            pltpu.sync_copy(src3_hbm.at[wid, i], idx4.at[r, 0])
            pltpu.sync_copy(dst3_hbm.at[wid, i], idx4.at[r, 1])
        else:
            pltpu.async_copy(src3_hbm.at[wid, i], idx4.at[r, 0], isem)
            pltpu.async_copy(dst3_hbm.at[wid, i], idx4.at[r, 1], isem)

    def _drain_idx(i):
        r = lax.rem(i, 4)
        pltpu.make_async_copy(src3_hbm.at[wid, i], idx4.at[r, 0],
                              isem).wait()
        pltpu.make_async_copy(dst3_hbm.at[wid, i], idx4.at[r, 1],
                              isem).wait()

    def _issue_data(i, b):
        base = base0 + i * C
        r = lax.rem(i, 4)
        pltpu.async_copy(eft_hbm.at[:, pl.ds(base, C)], ef2.at[b],
                         lsem.at[b])
        pltpu.async_copy(ts_hbm.at[idx4.at[r, 0]], gs2.at[b], gsem.at[b])
        pltpu.async_copy(td_hbm.at[idx4.at[r, 1]], gd2.at[b], gsem.at[b])

    def _drain_data(i, b):
        base = base0 + i * C
        r = lax.rem(i, 4)
        pltpu.make_async_copy(eft_hbm.at[:, pl.ds(base, C)], ef2.at[b],
                              lsem.at[b]).wait()
        pltpu.make_async_copy(ts_hbm.at[idx4.at[r, 0]], gs2.at[b],
                              gsem.at[b]).wait()
        pltpu.make_async_copy(td_hbm.at[idx4.at[r, 1]], gd2.at[b],
                              gsem.at[b]).wait()

    def _drain_out(j, b):
        base = base0 + j * C
        pltpu.make_async_copy(pv2.at[b], den_sh.at[idx4.at[lax.rem(j, 4), 1]],
                              ssem.at[b]).wait()
        pltpu.make_async_copy(pv2.at[b], p_hbm.at[pl.ds(base, C)],
                              wsem.at[b]).wait()

    _issue_idx(0, sync=True)
    _issue_data(0, 0)
    _issue_idx(1)

    @pl.loop(0, NCHUNK)
    def _chunk(i):
        b = lax.rem(i, 2)

        @pl.when(i + 1 < NCHUNK)
        def _():
            _drain_idx(i + 1)
            _issue_data(i + 1, 1 - b)

        @pl.when(i >= 2)
        def _():
            _drain_out(i - 2, b)

        @pl.when(i + 2 < NCHUNK)
        def _():
            _issue_idx(i + 2)

        _drain_data(i, b)

        w0 = wtv[0, :]
        w1 = wtv[1, :]
        w2 = wtv[2, :]
        w3 = wtv[3, :]

        @pl.loop(0, C, step=16)
        def _grp(c0):
            v0 = ef2[b, 0, pl.ds(c0, 16)]
            v1 = ef2[b, 1, pl.ds(c0, 16)]
            v2 = ef2[b, 2, pl.ds(c0, 16)]
            v3 = ef2[b, 3, pl.ds(c0, 16)]
            for l in range(16):
                c = c0 + l
                et = (_bcast_lane(v0, l) * w0 + _bcast_lane(v1, l) * w1
                      + _bcast_lane(v2, l) * w2 + _bcast_lane(v3, l) * w3)
                a = gs2[b, c, :] + gd2[b, c, :] + et
                a = jnp.where(a >= 0.0, a, a * 0.2)
                pv2[b, c, :] = jnp.exp(a)

        base = base0 + i * C
        pltpu.async_copy(pv2.at[b], den_sh.at[idx4.at[lax.rem(i, 4), 1]],
                         ssem.at[b], add=True)
        pltpu.async_copy(pv2.at[b], p_hbm.at[pl.ds(base, C)], wsem.at[b])

    _drain_out(NCHUNK - 2, lax.rem(NCHUNK - 2, 2))
    _drain_out(NCHUNK - 1, lax.rem(NCHUNK - 1, 2))

    plsc.subcore_barrier()

    @pl.loop(sid, NROWCHUNK, step=NS)
    def _wb(j):
        pltpu.sync_copy(den_sh.at[pl.ds(j * C, C)],
                        den_hbm.at[cid, pl.ds(j * C, C)])


def _den_pass(ts16, td16, eft, wt, src3, dst3):
    mesh = plsc.VectorSubcoreMesh(core_axis_name="c", subcore_axis_name="s")
    f = pl.kernel(
        _den_pass_body,
        out_type=(
            jax.ShapeDtypeStruct((E, 16), jnp.float32),
            jax.ShapeDtypeStruct((NC, N, 16), jnp.float32),
        ),
        mesh=mesh,
        compiler_params=pltpu.CompilerParams(use_tc_tiling_on_sc=False),
        scratch_types=[
            pltpu.VMEM((4, 2, C), jnp.int32),
            pltpu.VMEM((2, C, 16), jnp.float32),
            pltpu.VMEM((2, C, 16), jnp.float32),
            pltpu.VMEM((2, EDGE_FEATS, C), jnp.float32),
            pltpu.VMEM((2, C, 16), jnp.float32),
            pltpu.VMEM((EDGE_FEATS, 16), jnp.float32),
            pltpu.VMEM_SHARED((N, 16), jnp.float32),
            pltpu.SemaphoreType.DMA,
            pltpu.SemaphoreType.DMA((2,)),
            pltpu.SemaphoreType.DMA((2,)),
            pltpu.SemaphoreType.DMA((2,)),
            pltpu.SemaphoreType.DMA((2,)),
        ],
    )
    return f(ts16, td16, eft, wt, src3, dst3)


def _combine_body(den_ref, o_ref):
    r = den_ref[0] + den_ref[1]
    o_ref[...] = 1.0 / (HEADS * jnp.maximum(r, 1e-12))


def _combine(den):
    bn = 1000
    return pl.pallas_call(
        _combine_body,
        grid=(N // bn,),
        in_specs=[pl.BlockSpec((NC, bn, 16), lambda i: (0, i, 0))],
        out_specs=pl.BlockSpec((bn, 16), lambda i: (i, 0)),
        out_shape=jax.ShapeDtypeStruct((N, 16), jnp.float32),
    )(den)


def _agg_pass_body(mn_hbm, di_hbm, etm_hbm, p_hbm, src3_hbm, dst3_hbm,
                   out_hbm,
                   idx4, srow2, gdi2, etm2, pp2, v2, acc_sh,
                   isem, gsem, lsem, ssem):
    cid = lax.axis_index("c")
    sid = lax.axis_index("s")
    wid = sid * NC + cid
    base0 = wid * EPT

    @pl.loop(0, C)
    def _zrow(r):
        v2[0, r, :] = jnp.zeros((16,), jnp.float32)

    @pl.loop(sid, NROWCHUNK, step=NS)
    def _zacc(j):
        pltpu.sync_copy(v2.at[0], acc_sh.at[pl.ds(j * C, C)])

    plsc.subcore_barrier()

    def _issue_idx(i, sync=False):
        r = lax.rem(i, 4)
        if sync:
            pltpu.sync_copy(src3_hbm.at[wid, i], idx4.at[r, 0])
            pltpu.sync_copy(dst3_hbm.at[wid, i], idx4.at[r, 1])
        else:
            pltpu.async_copy(src3_hbm.at[wid, i], idx4.at[r, 0], isem)
            pltpu.async_copy(dst3_hbm.at[wid, i], idx4.at[r, 1], isem)

    def _drain_idx(i):
        r = lax.rem(i, 4)
        pltpu.make_async_copy(src3_hbm.at[wid, i], idx4.at[r, 0],
                              isem).wait()
        pltpu.make_async_copy(dst3_hbm.at[wid, i], idx4.at[r, 1],
                              isem).wait()

    def _issue_data(i, b):
        base = base0 + i * C
        r = lax.rem(i, 4)
        pltpu.async_copy(etm_hbm.at[pl.ds(base, C)], etm2.at[b], lsem.at[b])
        pltpu.async_copy(p_hbm.at[pl.ds(base, C)], pp2.at[b], lsem.at[b])
        pltpu.async_copy(mn_hbm.at[idx4.at[r, 0]], srow2.at[b], gsem.at[b])
        pltpu.async_copy(di_hbm.at[idx4.at[r, 1]], gdi2.at[b], gsem.at[b])

    def _drain_data(i, b):
        base = base0 + i * C
        r = lax.rem(i, 4)
        pltpu.make_async_copy(etm_hbm.at[pl.ds(base, C)], etm2.at[b],
                              lsem.at[b]).wait()
        pltpu.make_async_copy(p_hbm.at[pl.ds(base, C)], pp2.at[b],
                              lsem.at[b]).wait()
        pltpu.make_async_copy(mn_hbm.at[idx4.at[r, 0]], srow2.at[b],
                              gsem.at[b]).wait()
        pltpu.make_async_copy(di_hbm.at[idx4.at[r, 1]], gdi2.at[b],
                              gsem.at[b]).wait()

    def _drain_scatter(j, b):
        pltpu.make_async_copy(v2.at[b], acc_sh.at[idx4.at[lax.rem(j, 4), 1]],
                              ssem.at[b]).wait()

    _issue_idx(0, sync=True)
    _issue_data(0, 0)
    _issue_idx(1)

    @pl.loop(0, NCHUNK)
    def _chunk(i):
        b = lax.rem(i, 2)

        @pl.when(i + 1 < NCHUNK)
        def _():
            _drain_idx(i + 1)
            _issue_data(i + 1, 1 - b)

        @pl.when(i >= 2)
        def _():
            _drain_scatter(i - 2, b)

        @pl.when(i + 2 < NCHUNK)
        def _():
            _issue_idx(i + 2)

        _drain_data(i, b)

        @pl.loop(0, C, unroll=2)
        def _edge(c):
            al = pp2[b, c, :] * gdi2[b, c, :]
            acc = (srow2[b, c, pl.ds(0, 16)]
                   + etm2[b, c, pl.ds(0, 16)]) * al[0]
            for k in range(1, HEADS):
                acc = acc + (srow2[b, c, pl.ds(16 * k, 16)]
                             + etm2[b, c, pl.ds(16 * k, 16)]) * al[k]
            v2[b, c, :] = acc

        pltpu.async_copy(v2.at[b], acc_sh.at[idx4.at[lax.rem(i, 4), 1]],
                         ssem.at[b], add=True)

    _drain_scatter(NCHUNK - 2, lax.rem(NCHUNK - 2, 2))
    _drain_scatter(NCHUNK - 1, lax.rem(NCHUNK - 1, 2))

    plsc.subcore_barrier()

    @pl.loop(sid, NROWCHUNK, step=NS)
    def _wb(j):
        pltpu.sync_copy(acc_sh.at[pl.ds(j * C, C)],
                        out_hbm.at[cid, pl.ds(j * C, C)])


def _agg_pass(mnode, dinv, etm, pbuf, src3, dst3):
    mesh = plsc.VectorSubcoreMesh(core_axis_name="c", subcore_axis_name="s")
    f = pl.kernel(
        _agg_pass_body,
        out_type=jax.ShapeDtypeStruct((NC, N, 16), jnp.float32),
        mesh=mesh,
        compiler_params=pltpu.CompilerParams(use_tc_tiling_on_sc=False),
        scratch_types=[
            pltpu.VMEM((4, 2, C), jnp.int32),
            pltpu.VMEM((2, C, 128), jnp.float32),
            pltpu.VMEM((2, C, 16), jnp.float32),
            pltpu.VMEM((2, C, 128), jnp.float32),
            pltpu.VMEM((2, C, 16), jnp.float32),
            pltpu.VMEM((2, C, 16), jnp.float32),
            pltpu.VMEM_SHARED((N, 16), jnp.float32),
            pltpu.SemaphoreType.DMA,
            pltpu.SemaphoreType.DMA((2,)),
            pltpu.SemaphoreType.DMA((2,)),
            pltpu.SemaphoreType.DMA((2,)),
        ],
    )
    return f(mnode, dinv, etm, pbuf, src3, dst3)


def _final_body(a_ref, o_ref):
    o_ref[...] = a_ref[0] + a_ref[1]


def _finalize(outp):
    bn = 1000
    return pl.pallas_call(
        _final_body,
        grid=(N // bn,),
        in_specs=[pl.BlockSpec((NC, bn, 16), lambda i: (0, i, 0))],
        out_specs=pl.BlockSpec((bn, 16), lambda i: (i, 0)),
        out_shape=jax.ShapeDtypeStruct((N, 16), jnp.float32),
    )(outp)


def kernel(h, edge_index, edge_feat, W_node, W_edge, attention_src,
           attention_dst, W_msg):
    f32 = jnp.float32
    wn3 = W_node.reshape(IN_FEATS, HEADS, OUT_FEATS)
    a_src = jnp.einsum("jhk,hk->jh", wn3, attention_src)
    a_dst = jnp.einsum("jhk,hk->jh", wn3, attention_dst)
    pad_n = jnp.zeros((IN_FEATS, 8), f32)
    pad_e = jnp.zeros((EDGE_FEATS, 8), f32)
    ws = jnp.concatenate([a_src, pad_n], axis=1)
    wd = jnp.concatenate([a_dst, pad_n], axis=1)
    wt = jnp.concatenate([W_edge, pad_e], axis=1)

    mnode, ts16, td16 = _node_tables(h, W_msg[:IN_FEATS], ws, wd)
    eft = edge_feat.T
    src3 = edge_index[0].reshape(NW, NCHUNK, C)
    dst3 = edge_index[1].reshape(NW, NCHUNK, C)
    pbuf, den = _den_pass(ts16, td16, eft, wt, src3, dst3)
    etm = _edge_table(eft, W_msg[IN_FEATS:])
    dinv = _combine(den)
    outp = _agg_pass(mnode, dinv, etm, pbuf, src3, dst3)
    return _finalize(outp)
